# baseline (speedup 1.0000x reference)
import functools

import jax
import jax.numpy as jnp
from jax import lax
from jax.experimental import pallas as pl
from jax.experimental.pallas import tpu as pltpu
from jax.experimental.pallas import tpu_sc as plsc

N = 10000
E = 320000
D_IN = 128
D_H = 256
D_OUT = 64

NC = 2
NS = 16
NW = NC * NS
K = 128
EROWS = 2560
E_PAD = EROWS * K
N_ROWS = 10240
ROWS_PER_TILE = N_ROWS // NS
N_FLAT = 10240
A_ROWS = EROWS // NW
C_ROWS = EROWS // NS
IB = 16
CH = 2048
CCHUNKS = E_PAD // NW // CH

_mesh = plsc.VectorSubcoreMesh(core_axis_name="c", subcore_axis_name="s")


def _agg1_body(x_hbm, srcm, dstm, zfeat, zdeg,
               s0a, s0b, degp,
               src_v, dst_v, rows0, rows1, deg_t,
               sem_g0, sem_g1, sem_s0, sem_s1,
               acc):
    c = lax.axis_index("c")
    s = lax.axis_index("s")
    wid = c * NS + s
    pltpu.sync_copy(zfeat, acc.at[pl.ds(s * ROWS_PER_TILE, ROWS_PER_TILE)])
    pltpu.sync_copy(zdeg, deg_t)
    base = wid * A_ROWS
    ones16 = jnp.ones((16,), jnp.float32)
    plsc.subcore_barrier()

    def blk(bi, carry):
        pltpu.sync_copy(srcm.at[pl.ds(base + bi * IB, IB)], src_v)
        pltpu.sync_copy(dstm.at[pl.ds(base + bi * IB, IB)], dst_v)
        for j in range(IB):
            for u in range(K // 16):
                d16 = dst_v[j, pl.ds(u * 16, 16)]
                plsc.addupdate_scatter(deg_t, [d16], ones16)
        pltpu.async_copy(x_hbm.at[src_v.at[0]], rows0, sem_g0)

        def pair(t, carry2):
            j0 = 2 * t
            j1 = j0 + 1
            @pl.when(t > 0)
            def _():
                pltpu.make_async_copy(rows1, acc.at[dst_v.at[j1]], sem_s1).wait()

            pltpu.async_copy(x_hbm.at[src_v.at[j1]], rows1, sem_g1)
            pltpu.make_async_copy(x_hbm.at[src_v.at[j0]], rows0, sem_g0).wait()
            pltpu.async_copy(rows0, acc.at[dst_v.at[j0]], sem_s0, add=True)
            pltpu.make_async_copy(x_hbm.at[src_v.at[j1]], rows1, sem_g1).wait()
            pltpu.async_copy(rows1, acc.at[dst_v.at[j1]], sem_s1, add=True)

            @pl.when(t < IB // 2 - 1)
            def _():
                pltpu.make_async_copy(rows0, acc.at[dst_v.at[j0]], sem_s0).wait()
                pltpu.async_copy(x_hbm.at[src_v.at[j0 + 2]], rows0, sem_g0)

            return carry2

        lax.fori_loop(0, IB // 2, pair, carry)
        pltpu.make_async_copy(rows0, acc.at[dst_v.at[0]], sem_s0).wait()
        pltpu.make_async_copy(rows1, acc.at[dst_v.at[0]], sem_s1).wait()
        return carry

    lax.fori_loop(0, A_ROWS // IB, blk, 0)
    pltpu.sync_copy(deg_t, degp.at[pl.ds(wid * N_FLAT, N_FLAT)])
    plsc.subcore_barrier()

    rsl = pl.ds(s * ROWS_PER_TILE, ROWS_PER_TILE)

    @pl.when(c == 0)
    def _():
        pltpu.sync_copy(acc.at[rsl], s0a.at[rsl])

    @pl.when(c == 1)
    def _():
        pltpu.sync_copy(acc.at[rsl], s0b.at[rsl])


_agg1 = functools.partial(
    pl.kernel,
    out_type=[
        jax.ShapeDtypeStruct((N_ROWS, D_IN), jnp.float32),
        jax.ShapeDtypeStruct((N_ROWS, D_IN), jnp.float32),
        jax.ShapeDtypeStruct((NW * N_FLAT,), jnp.float32),
    ],
    mesh=_mesh,
    scratch_types=[
        pltpu.VMEM((IB, K), jnp.int32),
        pltpu.VMEM((IB, K), jnp.int32),
        pltpu.VMEM((K, D_IN), jnp.float32),
        pltpu.VMEM((K, D_IN), jnp.float32),
        pltpu.VMEM((N_FLAT,), jnp.float32),
        pltpu.SemaphoreType.DMA,
        pltpu.SemaphoreType.DMA,
        pltpu.SemaphoreType.DMA,
        pltpu.SemaphoreType.DMA,
        pltpu.VMEM_SHARED((N_ROWS, D_IN), jnp.float32),
    ],
    compiler_params=pltpu.CompilerParams(needs_layout_passes=False),
)(_agg1_body)


def _cker_body(w_hbm, srcf, dstf, zdeg, cp,
               w_t, c_t, s0_v, s1_v, d0_v, d1_v, sem0, sem1):
    c = lax.axis_index("c")
    s = lax.axis_index("s")
    wid = c * NS + s
    base = wid * (CCHUNKS * CH)
    pltpu.sync_copy(w_hbm, w_t)
    pltpu.sync_copy(zdeg, c_t)
    pltpu.async_copy(srcf.at[pl.ds(base, CH)], s0_v, sem0)
    pltpu.async_copy(dstf.at[pl.ds(base, CH)], d0_v, sem0)
    for bi in range(CCHUNKS):
        cur_s, cur_d = (s0_v, d0_v) if bi % 2 == 0 else (s1_v, d1_v)
        nxt_s, nxt_d = (s1_v, d1_v) if bi % 2 == 0 else (s0_v, d0_v)
        sem_cur = sem0 if bi % 2 == 0 else sem1
        sem_nxt = sem1 if bi % 2 == 0 else sem0
        off = base + bi * CH
        pltpu.make_async_copy(srcf.at[pl.ds(off, CH)], cur_s, sem_cur).wait()
        pltpu.make_async_copy(dstf.at[pl.ds(off, CH)], cur_d, sem_cur).wait()
        if bi + 1 < CCHUNKS:
            noff = base + (bi + 1) * CH
            pltpu.async_copy(srcf.at[pl.ds(noff, CH)], nxt_s, sem_nxt)
            pltpu.async_copy(dstf.at[pl.ds(noff, CH)], nxt_d, sem_nxt)
        for u in range(CH // 16):
            s16 = cur_s[pl.ds(u * 16, 16)]
            d16 = cur_d[pl.ds(u * 16, 16)]
            w16 = plsc.load_gather(w_t, [d16])
            plsc.addupdate_scatter(c_t, [s16], w16)
    pltpu.sync_copy(c_t, cp.at[pl.ds(wid * N_FLAT, N_FLAT)])


_cker = functools.partial(
    pl.kernel,
    out_type=jax.ShapeDtypeStruct((NW * N_FLAT,), jnp.float32),
    mesh=_mesh,
    scratch_types=[
        pltpu.VMEM((N_FLAT,), jnp.float32),
        pltpu.VMEM((N_FLAT,), jnp.float32),
        pltpu.VMEM((CH,), jnp.int32),
        pltpu.VMEM((CH,), jnp.int32),
        pltpu.VMEM((CH,), jnp.int32),
        pltpu.VMEM((CH,), jnp.int32),
        pltpu.SemaphoreType.DMA,
        pltpu.SemaphoreType.DMA,
    ],
    compiler_params=pltpu.CompilerParams(needs_layout_passes=False),
)(_cker_body)


def _agg2_body(h1a, h1b, srcm, dstm, zfeat,
               s1a, s1b,
               src_v, dst_v, rows0, rows1,
               sem_g0, sem_g1, sem_s0, sem_s1,
               acc):
    c = lax.axis_index("c")
    s = lax.axis_index("s")
    pltpu.sync_copy(zfeat, acc.at[pl.ds(s * ROWS_PER_TILE, ROWS_PER_TILE)])
    base = s * C_ROWS
    plsc.subcore_barrier()

    def gather_feat(j, buf, sem):
        @pl.when(c == 0)
        def _():
            pltpu.async_copy(h1a.at[src_v.at[j]], buf, sem)

        @pl.when(c == 1)
        def _():
            pltpu.async_copy(h1b.at[src_v.at[j]], buf, sem)

    def wait_feat(j, buf, sem):
        pltpu.make_async_copy(h1a.at[src_v.at[j]], buf, sem).wait()

    def blk(bi, carry):
        pltpu.sync_copy(srcm.at[pl.ds(base + bi * IB, IB)], src_v)
        pltpu.sync_copy(dstm.at[pl.ds(base + bi * IB, IB)], dst_v)
        gather_feat(0, rows0, sem_g0)

        def pair(t, carry2):
            j0 = 2 * t
            j1 = j0 + 1

            @pl.when(t > 0)
            def _():
                pltpu.make_async_copy(rows1, acc.at[dst_v.at[j1]], sem_s1).wait()

            gather_feat(j1, rows1, sem_g1)
            wait_feat(j0, rows0, sem_g0)
            pltpu.async_copy(rows0, acc.at[dst_v.at[j0]], sem_s0, add=True)
            wait_feat(j1, rows1, sem_g1)
            pltpu.async_copy(rows1, acc.at[dst_v.at[j1]], sem_s1, add=True)

            @pl.when(t < IB // 2 - 1)
            def _():
                pltpu.make_async_copy(rows0, acc.at[dst_v.at[j0]], sem_s0).wait()
                gather_feat(j0 + 2, rows0, sem_g0)

            return carry2

        lax.fori_loop(0, IB // 2, pair, carry)
        pltpu.make_async_copy(rows0, acc.at[dst_v.at[0]], sem_s0).wait()
        pltpu.make_async_copy(rows1, acc.at[dst_v.at[0]], sem_s1).wait()
        return carry

    lax.fori_loop(0, C_ROWS // IB, blk, 0)
    plsc.subcore_barrier()

    rsl = pl.ds(s * ROWS_PER_TILE, ROWS_PER_TILE)

    @pl.when(c == 0)
    def _():
        pltpu.sync_copy(acc.at[rsl], s1a.at[rsl])

    @pl.when(c == 1)
    def _():
        pltpu.sync_copy(acc.at[rsl], s1b.at[rsl])


_agg2 = functools.partial(
    pl.kernel,
    out_type=[
        jax.ShapeDtypeStruct((N_ROWS, D_IN), jnp.float32),
        jax.ShapeDtypeStruct((N_ROWS, D_IN), jnp.float32),
    ],
    mesh=_mesh,
    scratch_types=[
        pltpu.VMEM((IB, K), jnp.int32),
        pltpu.VMEM((IB, K), jnp.int32),
        pltpu.VMEM((K, D_IN), jnp.float32),
        pltpu.VMEM((K, D_IN), jnp.float32),
        pltpu.SemaphoreType.DMA,
        pltpu.SemaphoreType.DMA,
        pltpu.SemaphoreType.DMA,
        pltpu.SemaphoreType.DMA,
        pltpu.VMEM_SHARED((N_ROWS, D_IN), jnp.float32),
    ],
)(_agg2_body)


RB = 400
GRID = N // RB


def _degw_body(degp_ref, w_ref):
    dsum = jnp.sum(degp_ref[...], axis=0)
    row = lax.broadcasted_iota(jnp.int32, (N_FLAT // 128, 128), 0)
    col = lax.broadcasted_iota(jnp.int32, (N_FLAT // 128, 128), 1)
    nid = row * 128 + col
    w_ref[...] = jnp.where(nid < N, 1.0 / jnp.maximum(dsum, 1.0), 0.0)


def _degw(degp):
    return pl.pallas_call(
        _degw_body,
        out_shape=jax.ShapeDtypeStruct((N_FLAT // 128, 128), jnp.float32),
    )(degp)


def _csum_body(cp_ref, c_ref):
    c_ref[...] = jnp.sum(cp_ref[...], axis=0)


def _csum(cp):
    return pl.pallas_call(
        _csum_body,
        out_shape=jax.ShapeDtypeStruct((N_FLAT // 128, 128), jnp.float32),
    )(cp)


def _dense1_body(x_ref, sa_ref, sb_ref, w_ref, ws_ref, wn_ref, b_ref,
                 z_ref, sum_ref, sq_ref):
    i = pl.program_id(0)
    hn = (sa_ref[...] + sb_ref[...]) * w_ref[...]
    z = (jnp.dot(x_ref[...], ws_ref[...], preferred_element_type=jnp.float32)
         + jnp.dot(hn, wn_ref[...], preferred_element_type=jnp.float32)
         + b_ref[...])
    z_ref[...] = z
    zs = jnp.sum(z, axis=0, keepdims=True)
    z2 = jnp.sum(z * z, axis=0, keepdims=True)

    @pl.when(i == 0)
    def _():
        sum_ref[...] = zs
        sq_ref[...] = z2

    @pl.when(i > 0)
    def _():
        sum_ref[...] += zs
        sq_ref[...] += z2


def _dense1(x, sa, sb, w_col, ws, wn, b):
    d_in = x.shape[1]
    return pl.pallas_call(
        _dense1_body,
        grid=(GRID,),
        in_specs=[
            pl.BlockSpec((RB, d_in), lambda i: (i, 0)),
            pl.BlockSpec((RB, d_in), lambda i: (i, 0)),
            pl.BlockSpec((RB, d_in), lambda i: (i, 0)),
            pl.BlockSpec((RB, 1), lambda i: (i, 0)),
            pl.BlockSpec((d_in, D_H), lambda i: (0, 0)),
            pl.BlockSpec((d_in, D_H), lambda i: (0, 0)),
            pl.BlockSpec((1, D_H), lambda i: (0, 0)),
        ],
        out_specs=[
            pl.BlockSpec((RB, D_H), lambda i: (i, 0)),
            pl.BlockSpec((1, D_H), lambda i: (0, 0)),
            pl.BlockSpec((1, D_H), lambda i: (0, 0)),
        ],
        out_shape=[
            jax.ShapeDtypeStruct((N, D_H), jnp.float32),
            jax.ShapeDtypeStruct((1, D_H), jnp.float32),
            jax.ShapeDtypeStruct((1, D_H), jnp.float32),
        ],
    )(x, sa, sb, w_col, ws, wn, b)


def _dense2_body(ha_ref, hb_ref, sa_ref, sb_ref, w_ref, ws_ref, wn_ref,
                 b_ref, z_ref, sum_ref, sq_ref):
    i = pl.program_id(0)
    r = w_ref[...]
    hna = sa_ref[...] * r
    hnb = sb_ref[...] * r
    ws = ws_ref[...]
    wn = wn_ref[...]
    z = (jnp.dot(ha_ref[...], ws[:D_IN, :], preferred_element_type=jnp.float32)
         + jnp.dot(hb_ref[...], ws[D_IN:, :], preferred_element_type=jnp.float32)
         + jnp.dot(hna, wn[:D_IN, :], preferred_element_type=jnp.float32)
         + jnp.dot(hnb, wn[D_IN:, :], preferred_element_type=jnp.float32)
         + b_ref[...])
    z_ref[...] = z
    zs = jnp.sum(z, axis=0, keepdims=True)
    z2 = jnp.sum(z * z, axis=0, keepdims=True)

    @pl.when(i == 0)
    def _():
        sum_ref[...] = zs
        sq_ref[...] = z2

    @pl.when(i > 0)
    def _():
        sum_ref[...] += zs
        sq_ref[...] += z2


def _dense2(ha, hb, sa, sb, w_col, ws, wn, b):
    return pl.pallas_call(
        _dense2_body,
        grid=(GRID,),
        in_specs=[
            pl.BlockSpec((RB, D_IN), lambda i: (i, 0)),
            pl.BlockSpec((RB, D_IN), lambda i: (i, 0)),
            pl.BlockSpec((RB, D_IN), lambda i: (i, 0)),
            pl.BlockSpec((RB, D_IN), lambda i: (i, 0)),
            pl.BlockSpec((RB, 1), lambda i: (i, 0)),
            pl.BlockSpec((D_H, D_H), lambda i: (0, 0)),
            pl.BlockSpec((D_H, D_H), lambda i: (0, 0)),
            pl.BlockSpec((1, D_H), lambda i: (0, 0)),
        ],
        out_specs=[
            pl.BlockSpec((RB, D_H), lambda i: (i, 0)),
            pl.BlockSpec((1, D_H), lambda i: (0, 0)),
            pl.BlockSpec((1, D_H), lambda i: (0, 0)),
        ],
        out_shape=[
            jax.ShapeDtypeStruct((N, D_H), jnp.float32),
            jax.ShapeDtypeStruct((1, D_H), jnp.float32),
            jax.ShapeDtypeStruct((1, D_H), jnp.float32),
        ],
    )(ha, hb, sa, sb, w_col, ws, wn, b)


def _bnrelu_split_body(z_ref, sum_ref, sq_ref, g_ref, be_ref, ha_ref, hb_ref):
    mu = sum_ref[...] * (1.0 / N)
    var = sq_ref[...] * (1.0 / N) - mu * mu
    inv = lax.rsqrt(var + 1e-5) * g_ref[...]
    h = jnp.maximum((z_ref[...] - mu) * inv + be_ref[...], 0.0)
    ha_ref[...] = h[:, :D_IN]
    hb_ref[...] = h[:, D_IN:]


def _bnrelu_split(z, zsum, zsq, gamma, beta):
    return pl.pallas_call(
        _bnrelu_split_body,
        grid=(GRID,),
        in_specs=[
            pl.BlockSpec((RB, D_H), lambda i: (i, 0)),
            pl.BlockSpec((1, D_H), lambda i: (0, 0)),
            pl.BlockSpec((1, D_H), lambda i: (0, 0)),
            pl.BlockSpec((1, D_H), lambda i: (0, 0)),
            pl.BlockSpec((1, D_H), lambda i: (0, 0)),
        ],
        out_specs=[
            pl.BlockSpec((RB, D_IN), lambda i: (i, 0)),
            pl.BlockSpec((RB, D_IN), lambda i: (i, 0)),
        ],
        out_shape=[
            jax.ShapeDtypeStruct((N_ROWS, D_IN), jnp.float32),
            jax.ShapeDtypeStruct((N_ROWS, D_IN), jnp.float32),
        ],
    )(z, zsum, zsq, gamma, beta)


def _final_body(z_ref, sum_ref, sq_ref, g_ref, be_ref, c_ref,
                ws2_ref, wn2_ref, b2_ref, wlin_ref, blin_ref,
                out_ref, s2_acc, t2_acc):
    i = pl.program_id(0)
    mu = sum_ref[...] * (1.0 / N)
    var = sq_ref[...] * (1.0 / N) - mu * mu
    inv = lax.rsqrt(var + 1e-5) * g_ref[...]
    h = jnp.maximum((z_ref[...] - mu) * inv + be_ref[...], 0.0)
    s2 = jnp.sum(h, axis=0, keepdims=True)
    t2 = jnp.sum(h * c_ref[...], axis=0, keepdims=True)

    @pl.when(i == 0)
    def _():
        s2_acc[...] = s2
        t2_acc[...] = t2

    @pl.when(i > 0)
    def _():
        s2_acc[...] += s2
        t2_acc[...] += t2

    @pl.when(i == GRID - 1)
    def _():
        hg = (jnp.dot(s2_acc[...], ws2_ref[...], preferred_element_type=jnp.float32)
              + jnp.dot(t2_acc[...], wn2_ref[...], preferred_element_type=jnp.float32)
              + float(N) * b2_ref[...])
        out_ref[...] = (jnp.dot(hg, wlin_ref[...], preferred_element_type=jnp.float32)
                        + blin_ref[...])


def _final(z, zsum, zsq, gamma, beta, c_col, ws2, wn2, b2, wlin, blin):
    return pl.pallas_call(
        _final_body,
        grid=(GRID,),
        in_specs=[
            pl.BlockSpec((RB, D_H), lambda i: (i, 0)),
            pl.BlockSpec((1, D_H), lambda i: (0, 0)),
            pl.BlockSpec((1, D_H), lambda i: (0, 0)),
            pl.BlockSpec((1, D_H), lambda i: (0, 0)),
            pl.BlockSpec((1, D_H), lambda i: (0, 0)),
            pl.BlockSpec((RB, 1), lambda i: (i, 0)),
            pl.BlockSpec((D_H, D_H), lambda i: (0, 0)),
            pl.BlockSpec((D_H, D_H), lambda i: (0, 0)),
            pl.BlockSpec((1, D_H), lambda i: (0, 0)),
            pl.BlockSpec((D_H, D_OUT), lambda i: (0, 0)),
            pl.BlockSpec((1, D_OUT), lambda i: (0, 0)),
        ],
        out_specs=pl.BlockSpec((1, D_OUT), lambda i: (0, 0)),
        out_shape=jax.ShapeDtypeStruct((1, D_OUT), jnp.float32),
        scratch_shapes=[
            pltpu.VMEM((1, D_H), jnp.float32),
            pltpu.VMEM((1, D_H), jnp.float32),
        ],
    )(z, zsum, zsq, gamma, beta, c_col, ws2, wn2, b2, wlin, blin)


def kernel(x, edge_index, W_self0, W_neigh0, b0, W_self1, W_neigh1, b1,
           W_self2, W_neigh2, b2, gamma0, beta0, gamma1, beta1, W_lin, b_lin):
    src = edge_index[0]
    dst = edge_index[1]
    pad = E_PAD - E
    srcf = jnp.concatenate([src, jnp.zeros((pad,), jnp.int32)])
    dstf = jnp.concatenate([dst, jnp.full((pad,), N, jnp.int32)])
    srcm = srcf.reshape(EROWS, K)
    dstm = dstf.reshape(EROWS, K)
    zfeat = jnp.zeros((ROWS_PER_TILE, D_IN), jnp.float32)
    zdeg = jnp.zeros((N_FLAT,), jnp.float32)

    s0a, s0b, degp = _agg1(x, srcm, dstm, zfeat, zdeg)
    w2 = _degw(degp.reshape(NW, N_FLAT // 128, 128))
    w_flat = w2.reshape(N_FLAT)
    w_col = w_flat[:N, None]
    cp = _cker(w_flat, srcf, dstf, zdeg)

    z1, z1s, z1q = _dense1(x, s0a, s0b, w_col, W_self0, W_neigh0, b0[None, :])
    h1a, h1b = _bnrelu_split(z1, z1s, z1q, gamma0[None, :], beta0[None, :])

    s1a, s1b = _agg2(h1a, h1b, srcm, dstm, zfeat)
    c2 = _csum(cp.reshape(NW, N_FLAT // 128, 128))
    c_col = c2.reshape(N_FLAT)[:N, None]

    z2, z2s, z2q = _dense2(h1a, h1b, s1a, s1b, w_col,
                           W_self1, W_neigh1, b1[None, :])
    out = _final(z2, z2s, z2q, gamma1[None, :], beta1[None, :], c_col,
                 W_self2, W_neigh2, b2[None, :], W_lin, b_lin[None, :])
    return out


# speedup vs baseline: 1.0481x; 1.0481x over previous
import functools

import jax
import jax.numpy as jnp
from jax import lax
from jax.experimental import pallas as pl
from jax.experimental.pallas import tpu as pltpu
from jax.experimental.pallas import tpu_sc as plsc

N = 10000
E = 320000
D_IN = 128
D_H = 256
D_OUT = 64

NC = 2
NS = 16
NW = NC * NS
K = 64
E_PAD = 327680
N_ROWS = 10240
ROWS_PER_TILE = N_ROWS // NS
N_FLAT = 10240
NBUF = 4
CH = 2048
GRP = CH // (NBUF * K)
A_CH = E_PAD // NW // CH
C_CH = E_PAD // NS // CH
CCHUNKS = E_PAD // NW // CH

_mesh = plsc.VectorSubcoreMesh(core_axis_name="c", subcore_axis_name="s")


def _agg1_body(x_hbm, srcf, dstf, zfeat, zdeg,
               s0a, s0b, degp,
               src_v, dst_v, r0, r1, r2, r3, deg_t,
               sg0, sg1, sg2, sg3, ss0, ss1, ss2, ss3,
               acc):
    c = lax.axis_index("c")
    s = lax.axis_index("s")
    wid = c * NS + s
    pltpu.sync_copy(zfeat, acc.at[pl.ds(s * ROWS_PER_TILE, ROWS_PER_TILE)])
    pltpu.sync_copy(zdeg, deg_t)
    base = wid * (A_CH * CH)
    ones16 = jnp.ones((16,), jnp.float32)
    bufs = (r0, r1, r2, r3)
    sgs = (sg0, sg1, sg2, sg3)
    sss = (ss0, ss1, ss2, ss3)
    plsc.subcore_barrier()

    def chunk(ci, carry):
        off = base + ci * CH
        pltpu.sync_copy(srcf.at[pl.ds(off, CH)], src_v)
        pltpu.sync_copy(dstf.at[pl.ds(off, CH)], dst_v)
        for u in range(CH // 16):
            d16 = dst_v[pl.ds(u * 16, 16)]
            plsc.addupdate_scatter(deg_t, [d16], ones16)
        for r in range(NBUF):
            pltpu.async_copy(
                x_hbm.at[src_v.at[pl.ds(r * K, K)]], bufs[r], sgs[r])

        def grp(g, carry2):
            gbase = g * (NBUF * K)
            for r in range(NBUF):
                eoff = gbase + r * K
                pltpu.make_async_copy(
                    x_hbm.at[src_v.at[pl.ds(eoff, K)]], bufs[r], sgs[r]).wait()
                pltpu.async_copy(
                    bufs[r], acc.at[dst_v.at[pl.ds(eoff, K)]], sss[r], add=True)

            @pl.when(g < GRP - 1)
            def _():
                for r in range(NBUF):
                    noff = gbase + NBUF * K + r * K
                    pltpu.make_async_copy(
                        bufs[r], acc.at[dst_v.at[pl.ds(0, K)]], sss[r]).wait()
                    pltpu.async_copy(
                        x_hbm.at[src_v.at[pl.ds(noff, K)]], bufs[r], sgs[r])

            return carry2

        lax.fori_loop(0, GRP, grp, carry)
        for r in range(NBUF):
            pltpu.make_async_copy(
                bufs[r], acc.at[dst_v.at[pl.ds(0, K)]], sss[r]).wait()
        return carry

    lax.fori_loop(0, A_CH, chunk, 0)
    pltpu.sync_copy(deg_t, degp.at[pl.ds(wid * N_FLAT, N_FLAT)])
    plsc.subcore_barrier()

    rsl = pl.ds(s * ROWS_PER_TILE, ROWS_PER_TILE)

    @pl.when(c == 0)
    def _():
        pltpu.sync_copy(acc.at[rsl], s0a.at[rsl])

    @pl.when(c == 1)
    def _():
        pltpu.sync_copy(acc.at[rsl], s0b.at[rsl])


_agg1 = functools.partial(
    pl.kernel,
    out_type=[
        jax.ShapeDtypeStruct((N_ROWS, D_IN), jnp.float32),
        jax.ShapeDtypeStruct((N_ROWS, D_IN), jnp.float32),
        jax.ShapeDtypeStruct((NW * N_FLAT,), jnp.float32),
    ],
    mesh=_mesh,
    scratch_types=[
        pltpu.VMEM((CH,), jnp.int32),
        pltpu.VMEM((CH,), jnp.int32),
        pltpu.VMEM((K, D_IN), jnp.float32),
        pltpu.VMEM((K, D_IN), jnp.float32),
        pltpu.VMEM((K, D_IN), jnp.float32),
        pltpu.VMEM((K, D_IN), jnp.float32),
        pltpu.VMEM((N_FLAT,), jnp.float32),
        pltpu.SemaphoreType.DMA,
        pltpu.SemaphoreType.DMA,
        pltpu.SemaphoreType.DMA,
        pltpu.SemaphoreType.DMA,
        pltpu.SemaphoreType.DMA,
        pltpu.SemaphoreType.DMA,
        pltpu.SemaphoreType.DMA,
        pltpu.SemaphoreType.DMA,
        pltpu.VMEM_SHARED((N_ROWS, D_IN), jnp.float32),
    ],
    compiler_params=pltpu.CompilerParams(needs_layout_passes=False),
)(_agg1_body)


def _cker_body(w_hbm, srcf, dstf, zdeg, cp,
               w_t, c_t, s0_v, s1_v, d0_v, d1_v, sem0, sem1):
    c = lax.axis_index("c")
    s = lax.axis_index("s")
    wid = c * NS + s
    base = wid * (CCHUNKS * CH)
    pltpu.sync_copy(w_hbm, w_t)
    pltpu.sync_copy(zdeg, c_t)
    pltpu.async_copy(srcf.at[pl.ds(base, CH)], s0_v, sem0)
    pltpu.async_copy(dstf.at[pl.ds(base, CH)], d0_v, sem0)
    for bi in range(CCHUNKS):
        cur_s, cur_d = (s0_v, d0_v) if bi % 2 == 0 else (s1_v, d1_v)
        nxt_s, nxt_d = (s1_v, d1_v) if bi % 2 == 0 else (s0_v, d0_v)
        sem_cur = sem0 if bi % 2 == 0 else sem1
        sem_nxt = sem1 if bi % 2 == 0 else sem0
        off = base + bi * CH
        pltpu.make_async_copy(srcf.at[pl.ds(off, CH)], cur_s, sem_cur).wait()
        pltpu.make_async_copy(dstf.at[pl.ds(off, CH)], cur_d, sem_cur).wait()
        if bi + 1 < CCHUNKS:
            noff = base + (bi + 1) * CH
            pltpu.async_copy(srcf.at[pl.ds(noff, CH)], nxt_s, sem_nxt)
            pltpu.async_copy(dstf.at[pl.ds(noff, CH)], nxt_d, sem_nxt)
        for u in range(CH // 16):
            s16 = cur_s[pl.ds(u * 16, 16)]
            d16 = cur_d[pl.ds(u * 16, 16)]
            w16 = plsc.load_gather(w_t, [d16])
            plsc.addupdate_scatter(c_t, [s16], w16)
    pltpu.sync_copy(c_t, cp.at[pl.ds(wid * N_FLAT, N_FLAT)])


_cker = functools.partial(
    pl.kernel,
    out_type=jax.ShapeDtypeStruct((NW * N_FLAT,), jnp.float32),
    mesh=_mesh,
    scratch_types=[
        pltpu.VMEM((N_FLAT,), jnp.float32),
        pltpu.VMEM((N_FLAT,), jnp.float32),
        pltpu.VMEM((CH,), jnp.int32),
        pltpu.VMEM((CH,), jnp.int32),
        pltpu.VMEM((CH,), jnp.int32),
        pltpu.VMEM((CH,), jnp.int32),
        pltpu.SemaphoreType.DMA,
        pltpu.SemaphoreType.DMA,
    ],
    compiler_params=pltpu.CompilerParams(needs_layout_passes=False),
)(_cker_body)


def _agg2_body(h1a, h1b, srcf, dstf, zfeat,
               s1a, s1b,
               src_v, dst_v, r0, r1, r2, r3,
               sg0, sg1, sg2, sg3, ss0, ss1, ss2, ss3,
               acc):
    c = lax.axis_index("c")
    s = lax.axis_index("s")
    pltpu.sync_copy(zfeat, acc.at[pl.ds(s * ROWS_PER_TILE, ROWS_PER_TILE)])
    base = s * (C_CH * CH)
    bufs = (r0, r1, r2, r3)
    sgs = (sg0, sg1, sg2, sg3)
    sss = (ss0, ss1, ss2, ss3)
    plsc.subcore_barrier()

    def gather_feat(eoff, buf, sem):
        @pl.when(c == 0)
        def _():
            pltpu.async_copy(h1a.at[src_v.at[pl.ds(eoff, K)]], buf, sem)

        @pl.when(c == 1)
        def _():
            pltpu.async_copy(h1b.at[src_v.at[pl.ds(eoff, K)]], buf, sem)

    def wait_feat(eoff, buf, sem):
        pltpu.make_async_copy(h1a.at[src_v.at[pl.ds(eoff, K)]], buf, sem).wait()

    def chunk(ci, carry):
        off = base + ci * CH
        pltpu.sync_copy(srcf.at[pl.ds(off, CH)], src_v)
        pltpu.sync_copy(dstf.at[pl.ds(off, CH)], dst_v)
        for r in range(NBUF):
            gather_feat(r * K, bufs[r], sgs[r])

        def grp(g, carry2):
            gbase = g * (NBUF * K)
            for r in range(NBUF):
                eoff = gbase + r * K
                wait_feat(eoff, bufs[r], sgs[r])
                pltpu.async_copy(
                    bufs[r], acc.at[dst_v.at[pl.ds(eoff, K)]], sss[r], add=True)

            @pl.when(g < GRP - 1)
            def _():
                for r in range(NBUF):
                    noff = gbase + NBUF * K + r * K
                    pltpu.make_async_copy(
                        bufs[r], acc.at[dst_v.at[pl.ds(0, K)]], sss[r]).wait()
                    gather_feat(noff, bufs[r], sgs[r])

            return carry2

        lax.fori_loop(0, GRP, grp, carry)
        for r in range(NBUF):
            pltpu.make_async_copy(
                bufs[r], acc.at[dst_v.at[pl.ds(0, K)]], sss[r]).wait()
        return carry

    lax.fori_loop(0, C_CH, chunk, 0)
    plsc.subcore_barrier()

    rsl = pl.ds(s * ROWS_PER_TILE, ROWS_PER_TILE)

    @pl.when(c == 0)
    def _():
        pltpu.sync_copy(acc.at[rsl], s1a.at[rsl])

    @pl.when(c == 1)
    def _():
        pltpu.sync_copy(acc.at[rsl], s1b.at[rsl])


_agg2 = functools.partial(
    pl.kernel,
    out_type=[
        jax.ShapeDtypeStruct((N_ROWS, D_IN), jnp.float32),
        jax.ShapeDtypeStruct((N_ROWS, D_IN), jnp.float32),
    ],
    mesh=_mesh,
    scratch_types=[
        pltpu.VMEM((CH,), jnp.int32),
        pltpu.VMEM((CH,), jnp.int32),
        pltpu.VMEM((K, D_IN), jnp.float32),
        pltpu.VMEM((K, D_IN), jnp.float32),
        pltpu.VMEM((K, D_IN), jnp.float32),
        pltpu.VMEM((K, D_IN), jnp.float32),
        pltpu.SemaphoreType.DMA,
        pltpu.SemaphoreType.DMA,
        pltpu.SemaphoreType.DMA,
        pltpu.SemaphoreType.DMA,
        pltpu.SemaphoreType.DMA,
        pltpu.SemaphoreType.DMA,
        pltpu.SemaphoreType.DMA,
        pltpu.SemaphoreType.DMA,
        pltpu.VMEM_SHARED((N_ROWS, D_IN), jnp.float32),
    ],
)(_agg2_body)


RB = 400
GRID = N // RB


def _degw_body(degp_ref, w_ref):
    dsum = jnp.sum(degp_ref[...], axis=0)
    row = lax.broadcasted_iota(jnp.int32, (N_FLAT // 128, 128), 0)
    col = lax.broadcasted_iota(jnp.int32, (N_FLAT // 128, 128), 1)
    nid = row * 128 + col
    w_ref[...] = jnp.where(nid < N, 1.0 / jnp.maximum(dsum, 1.0), 0.0)


def _degw(degp):
    return pl.pallas_call(
        _degw_body,
        out_shape=jax.ShapeDtypeStruct((N_FLAT // 128, 128), jnp.float32),
    )(degp)


def _csum_body(cp_ref, c_ref):
    c_ref[...] = jnp.sum(cp_ref[...], axis=0)


def _csum(cp):
    return pl.pallas_call(
        _csum_body,
        out_shape=jax.ShapeDtypeStruct((N_FLAT // 128, 128), jnp.float32),
    )(cp)


def _dense1_body(x_ref, sa_ref, sb_ref, w_ref, ws_ref, wn_ref, b_ref,
                 z_ref, sum_ref, sq_ref):
    i = pl.program_id(0)
    hn = (sa_ref[...] + sb_ref[...]) * w_ref[...]
    z = (jnp.dot(x_ref[...], ws_ref[...], preferred_element_type=jnp.float32)
         + jnp.dot(hn, wn_ref[...], preferred_element_type=jnp.float32)
         + b_ref[...])
    z_ref[...] = z
    zs = jnp.sum(z, axis=0, keepdims=True)
    z2 = jnp.sum(z * z, axis=0, keepdims=True)

    @pl.when(i == 0)
    def _():
        sum_ref[...] = zs
        sq_ref[...] = z2

    @pl.when(i > 0)
    def _():
        sum_ref[...] += zs
        sq_ref[...] += z2


def _dense1(x, sa, sb, w_col, ws, wn, b):
    d_in = x.shape[1]
    return pl.pallas_call(
        _dense1_body,
        grid=(GRID,),
        in_specs=[
            pl.BlockSpec((RB, d_in), lambda i: (i, 0)),
            pl.BlockSpec((RB, d_in), lambda i: (i, 0)),
            pl.BlockSpec((RB, d_in), lambda i: (i, 0)),
            pl.BlockSpec((RB, 1), lambda i: (i, 0)),
            pl.BlockSpec((d_in, D_H), lambda i: (0, 0)),
            pl.BlockSpec((d_in, D_H), lambda i: (0, 0)),
            pl.BlockSpec((1, D_H), lambda i: (0, 0)),
        ],
        out_specs=[
            pl.BlockSpec((RB, D_H), lambda i: (i, 0)),
            pl.BlockSpec((1, D_H), lambda i: (0, 0)),
            pl.BlockSpec((1, D_H), lambda i: (0, 0)),
        ],
        out_shape=[
            jax.ShapeDtypeStruct((N, D_H), jnp.float32),
            jax.ShapeDtypeStruct((1, D_H), jnp.float32),
            jax.ShapeDtypeStruct((1, D_H), jnp.float32),
        ],
    )(x, sa, sb, w_col, ws, wn, b)


def _dense2_body(ha_ref, hb_ref, sa_ref, sb_ref, w_ref, ws_ref, wn_ref,
                 b_ref, z_ref, sum_ref, sq_ref):
    i = pl.program_id(0)
    r = w_ref[...]
    hna = sa_ref[...] * r
    hnb = sb_ref[...] * r
    ws = ws_ref[...]
    wn = wn_ref[...]
    z = (jnp.dot(ha_ref[...], ws[:D_IN, :], preferred_element_type=jnp.float32)
         + jnp.dot(hb_ref[...], ws[D_IN:, :], preferred_element_type=jnp.float32)
         + jnp.dot(hna, wn[:D_IN, :], preferred_element_type=jnp.float32)
         + jnp.dot(hnb, wn[D_IN:, :], preferred_element_type=jnp.float32)
         + b_ref[...])
    z_ref[...] = z
    zs = jnp.sum(z, axis=0, keepdims=True)
    z2 = jnp.sum(z * z, axis=0, keepdims=True)

    @pl.when(i == 0)
    def _():
        sum_ref[...] = zs
        sq_ref[...] = z2

    @pl.when(i > 0)
    def _():
        sum_ref[...] += zs
        sq_ref[...] += z2


def _dense2(ha, hb, sa, sb, w_col, ws, wn, b):
    return pl.pallas_call(
        _dense2_body,
        grid=(GRID,),
        in_specs=[
            pl.BlockSpec((RB, D_IN), lambda i: (i, 0)),
            pl.BlockSpec((RB, D_IN), lambda i: (i, 0)),
            pl.BlockSpec((RB, D_IN), lambda i: (i, 0)),
            pl.BlockSpec((RB, D_IN), lambda i: (i, 0)),
            pl.BlockSpec((RB, 1), lambda i: (i, 0)),
            pl.BlockSpec((D_H, D_H), lambda i: (0, 0)),
            pl.BlockSpec((D_H, D_H), lambda i: (0, 0)),
            pl.BlockSpec((1, D_H), lambda i: (0, 0)),
        ],
        out_specs=[
            pl.BlockSpec((RB, D_H), lambda i: (i, 0)),
            pl.BlockSpec((1, D_H), lambda i: (0, 0)),
            pl.BlockSpec((1, D_H), lambda i: (0, 0)),
        ],
        out_shape=[
            jax.ShapeDtypeStruct((N, D_H), jnp.float32),
            jax.ShapeDtypeStruct((1, D_H), jnp.float32),
            jax.ShapeDtypeStruct((1, D_H), jnp.float32),
        ],
    )(ha, hb, sa, sb, w_col, ws, wn, b)


def _bnrelu_split_body(z_ref, sum_ref, sq_ref, g_ref, be_ref, ha_ref, hb_ref):
    mu = sum_ref[...] * (1.0 / N)
    var = sq_ref[...] * (1.0 / N) - mu * mu
    inv = lax.rsqrt(var + 1e-5) * g_ref[...]
    h = jnp.maximum((z_ref[...] - mu) * inv + be_ref[...], 0.0)
    ha_ref[...] = h[:, :D_IN]
    hb_ref[...] = h[:, D_IN:]


def _bnrelu_split(z, zsum, zsq, gamma, beta):
    return pl.pallas_call(
        _bnrelu_split_body,
        grid=(GRID,),
        in_specs=[
            pl.BlockSpec((RB, D_H), lambda i: (i, 0)),
            pl.BlockSpec((1, D_H), lambda i: (0, 0)),
            pl.BlockSpec((1, D_H), lambda i: (0, 0)),
            pl.BlockSpec((1, D_H), lambda i: (0, 0)),
            pl.BlockSpec((1, D_H), lambda i: (0, 0)),
        ],
        out_specs=[
            pl.BlockSpec((RB, D_IN), lambda i: (i, 0)),
            pl.BlockSpec((RB, D_IN), lambda i: (i, 0)),
        ],
        out_shape=[
            jax.ShapeDtypeStruct((N_ROWS, D_IN), jnp.float32),
            jax.ShapeDtypeStruct((N_ROWS, D_IN), jnp.float32),
        ],
    )(z, zsum, zsq, gamma, beta)


def _final_body(z_ref, sum_ref, sq_ref, g_ref, be_ref, c_ref,
                ws2_ref, wn2_ref, b2_ref, wlin_ref, blin_ref,
                out_ref, s2_acc, t2_acc):
    i = pl.program_id(0)
    mu = sum_ref[...] * (1.0 / N)
    var = sq_ref[...] * (1.0 / N) - mu * mu
    inv = lax.rsqrt(var + 1e-5) * g_ref[...]
    h = jnp.maximum((z_ref[...] - mu) * inv + be_ref[...], 0.0)
    s2 = jnp.sum(h, axis=0, keepdims=True)
    t2 = jnp.sum(h * c_ref[...], axis=0, keepdims=True)

    @pl.when(i == 0)
    def _():
        s2_acc[...] = s2
        t2_acc[...] = t2

    @pl.when(i > 0)
    def _():
        s2_acc[...] += s2
        t2_acc[...] += t2

    @pl.when(i == GRID - 1)
    def _():
        hg = (jnp.dot(s2_acc[...], ws2_ref[...], preferred_element_type=jnp.float32)
              + jnp.dot(t2_acc[...], wn2_ref[...], preferred_element_type=jnp.float32)
              + float(N) * b2_ref[...])
        out_ref[...] = (jnp.dot(hg, wlin_ref[...], preferred_element_type=jnp.float32)
                        + blin_ref[...])


def _final(z, zsum, zsq, gamma, beta, c_col, ws2, wn2, b2, wlin, blin):
    return pl.pallas_call(
        _final_body,
        grid=(GRID,),
        in_specs=[
            pl.BlockSpec((RB, D_H), lambda i: (i, 0)),
            pl.BlockSpec((1, D_H), lambda i: (0, 0)),
            pl.BlockSpec((1, D_H), lambda i: (0, 0)),
            pl.BlockSpec((1, D_H), lambda i: (0, 0)),
            pl.BlockSpec((1, D_H), lambda i: (0, 0)),
            pl.BlockSpec((RB, 1), lambda i: (i, 0)),
            pl.BlockSpec((D_H, D_H), lambda i: (0, 0)),
            pl.BlockSpec((D_H, D_H), lambda i: (0, 0)),
            pl.BlockSpec((1, D_H), lambda i: (0, 0)),
            pl.BlockSpec((D_H, D_OUT), lambda i: (0, 0)),
            pl.BlockSpec((1, D_OUT), lambda i: (0, 0)),
        ],
        out_specs=pl.BlockSpec((1, D_OUT), lambda i: (0, 0)),
        out_shape=jax.ShapeDtypeStruct((1, D_OUT), jnp.float32),
        scratch_shapes=[
            pltpu.VMEM((1, D_H), jnp.float32),
            pltpu.VMEM((1, D_H), jnp.float32),
        ],
    )(z, zsum, zsq, gamma, beta, c_col, ws2, wn2, b2, wlin, blin)


def kernel(x, edge_index, W_self0, W_neigh0, b0, W_self1, W_neigh1, b1,
           W_self2, W_neigh2, b2, gamma0, beta0, gamma1, beta1, W_lin, b_lin):
    src = edge_index[0]
    dst = edge_index[1]
    pad = E_PAD - E
    srcf = jnp.concatenate([src, jnp.zeros((pad,), jnp.int32)])
    dstf = jnp.concatenate([dst, jnp.full((pad,), N, jnp.int32)])
    zfeat = jnp.zeros((ROWS_PER_TILE, D_IN), jnp.float32)
    zdeg = jnp.zeros((N_FLAT,), jnp.float32)

    s0a, s0b, degp = _agg1(x, srcf, dstf, zfeat, zdeg)
    w2 = _degw(degp.reshape(NW, N_FLAT // 128, 128))
    w_flat = w2.reshape(N_FLAT)
    w_col = w_flat[:N, None]
    cp = _cker(w_flat, srcf, dstf, zdeg)

    z1, z1s, z1q = _dense1(x, s0a, s0b, w_col, W_self0, W_neigh0, b0[None, :])
    h1a, h1b = _bnrelu_split(z1, z1s, z1q, gamma0[None, :], beta0[None, :])

    s1a, s1b = _agg2(h1a, h1b, srcf, dstf, zfeat)
    c2 = _csum(cp.reshape(NW, N_FLAT // 128, 128))
    c_col = c2.reshape(N_FLAT)[:N, None]

    z2, z2s, z2q = _dense2(h1a, h1b, s1a, s1b, w_col,
                           W_self1, W_neigh1, b1[None, :])
    out = _final(z2, z2s, z2q, gamma1[None, :], beta1[None, :], c_col,
                 W_self2, W_neigh2, b2[None, :], W_lin, b_lin[None, :])
    return out


# speedup vs baseline: 1.0638x; 1.0150x over previous
import functools

import jax
import jax.numpy as jnp
from jax import lax
from jax.experimental import pallas as pl
from jax.experimental.pallas import tpu as pltpu
from jax.experimental.pallas import tpu_sc as plsc

N = 10000
E = 320000
D_IN = 128
D_H = 256
D_OUT = 64

NC = 2
NS = 16
NW = NC * NS
K = 64
E_PAD = 327680
N_ROWS = 10240
ROWS_PER_TILE = N_ROWS // NS
N_FLAT = 10240
NBUF = 4
CH = 2048
GRP = CH // (NBUF * K)
A_CH = E_PAD // NW // CH
CH2 = 4096
GRP2 = CH2 // (NBUF * K)
C_CH = E_PAD // NS // CH2
CCHUNKS = E_PAD // NW // CH

_mesh = plsc.VectorSubcoreMesh(core_axis_name="c", subcore_axis_name="s")


def _agg1_body(x_hbm, srcf, dstf, zfeat, zdeg,
               s0a, s0b, degp,
               src_v, dst_v, r0, r1, r2, r3, deg_t,
               sg0, sg1, sg2, sg3, ss0, ss1, ss2, ss3,
               acc):
    c = lax.axis_index("c")
    s = lax.axis_index("s")
    wid = c * NS + s
    pltpu.sync_copy(zfeat, acc.at[pl.ds(s * ROWS_PER_TILE, ROWS_PER_TILE)])
    pltpu.sync_copy(zdeg, deg_t)
    base = wid * (A_CH * CH)
    ones16 = jnp.ones((16,), jnp.float32)
    bufs = (r0, r1, r2, r3)
    sgs = (sg0, sg1, sg2, sg3)
    sss = (ss0, ss1, ss2, ss3)
    plsc.subcore_barrier()

    def chunk(ci, carry):
        off = base + ci * CH
        pltpu.sync_copy(srcf.at[pl.ds(off, CH)], src_v)
        pltpu.sync_copy(dstf.at[pl.ds(off, CH)], dst_v)
        for r in range(NBUF):
            pltpu.async_copy(
                x_hbm.at[src_v.at[pl.ds(r * K, K)]], bufs[r], sgs[r])
        for u in range(CH // 16):
            d16 = dst_v[pl.ds(u * 16, 16)]
            plsc.addupdate_scatter(deg_t, [d16], ones16)

        def grp(g, carry2):
            gbase = g * (NBUF * K)
            for r in range(NBUF):
                eoff = gbase + r * K
                pltpu.make_async_copy(
                    x_hbm.at[src_v.at[pl.ds(eoff, K)]], bufs[r], sgs[r]).wait()
                pltpu.async_copy(
                    bufs[r], acc.at[dst_v.at[pl.ds(eoff, K)]], sss[r], add=True)

            @pl.when(g < GRP - 1)
            def _():
                for r in range(NBUF):
                    noff = gbase + NBUF * K + r * K
                    pltpu.make_async_copy(
                        bufs[r], acc.at[dst_v.at[pl.ds(0, K)]], sss[r]).wait()
                    pltpu.async_copy(
                        x_hbm.at[src_v.at[pl.ds(noff, K)]], bufs[r], sgs[r])

            return carry2

        lax.fori_loop(0, GRP, grp, carry)
        for r in range(NBUF):
            pltpu.make_async_copy(
                bufs[r], acc.at[dst_v.at[pl.ds(0, K)]], sss[r]).wait()
        return carry

    lax.fori_loop(0, A_CH, chunk, 0)
    pltpu.sync_copy(deg_t, degp.at[pl.ds(wid * N_FLAT, N_FLAT)])
    plsc.subcore_barrier()

    rsl = pl.ds(s * ROWS_PER_TILE, ROWS_PER_TILE)

    @pl.when(c == 0)
    def _():
        pltpu.sync_copy(acc.at[rsl], s0a.at[rsl])

    @pl.when(c == 1)
    def _():
        pltpu.sync_copy(acc.at[rsl], s0b.at[rsl])


_agg1 = functools.partial(
    pl.kernel,
    out_type=[
        jax.ShapeDtypeStruct((N_ROWS, D_IN), jnp.float32),
        jax.ShapeDtypeStruct((N_ROWS, D_IN), jnp.float32),
        jax.ShapeDtypeStruct((NW * N_FLAT,), jnp.float32),
    ],
    mesh=_mesh,
    scratch_types=[
        pltpu.VMEM((CH,), jnp.int32),
        pltpu.VMEM((CH,), jnp.int32),
        pltpu.VMEM((K, D_IN), jnp.float32),
        pltpu.VMEM((K, D_IN), jnp.float32),
        pltpu.VMEM((K, D_IN), jnp.float32),
        pltpu.VMEM((K, D_IN), jnp.float32),
        pltpu.VMEM((N_FLAT,), jnp.float32),
        pltpu.SemaphoreType.DMA,
        pltpu.SemaphoreType.DMA,
        pltpu.SemaphoreType.DMA,
        pltpu.SemaphoreType.DMA,
        pltpu.SemaphoreType.DMA,
        pltpu.SemaphoreType.DMA,
        pltpu.SemaphoreType.DMA,
        pltpu.SemaphoreType.DMA,
        pltpu.VMEM_SHARED((N_ROWS, D_IN), jnp.float32),
    ],
    compiler_params=pltpu.CompilerParams(needs_layout_passes=False),
)(_agg1_body)


def _cker_body(w_hbm, srcf, dstf, zdeg, cp,
               w_t, c_t, s0_v, s1_v, d0_v, d1_v, sem0, sem1):
    c = lax.axis_index("c")
    s = lax.axis_index("s")
    wid = c * NS + s
    base = wid * (CCHUNKS * CH)
    pltpu.sync_copy(w_hbm, w_t)
    pltpu.sync_copy(zdeg, c_t)
    pltpu.async_copy(srcf.at[pl.ds(base, CH)], s0_v, sem0)
    pltpu.async_copy(dstf.at[pl.ds(base, CH)], d0_v, sem0)
    for bi in range(CCHUNKS):
        cur_s, cur_d = (s0_v, d0_v) if bi % 2 == 0 else (s1_v, d1_v)
        nxt_s, nxt_d = (s1_v, d1_v) if bi % 2 == 0 else (s0_v, d0_v)
        sem_cur = sem0 if bi % 2 == 0 else sem1
        sem_nxt = sem1 if bi % 2 == 0 else sem0
        off = base + bi * CH
        pltpu.make_async_copy(srcf.at[pl.ds(off, CH)], cur_s, sem_cur).wait()
        pltpu.make_async_copy(dstf.at[pl.ds(off, CH)], cur_d, sem_cur).wait()
        if bi + 1 < CCHUNKS:
            noff = base + (bi + 1) * CH
            pltpu.async_copy(srcf.at[pl.ds(noff, CH)], nxt_s, sem_nxt)
            pltpu.async_copy(dstf.at[pl.ds(noff, CH)], nxt_d, sem_nxt)
        for u in range(CH // 16):
            s16 = cur_s[pl.ds(u * 16, 16)]
            d16 = cur_d[pl.ds(u * 16, 16)]
            w16 = plsc.load_gather(w_t, [d16])
            plsc.addupdate_scatter(c_t, [s16], w16)
    pltpu.sync_copy(c_t, cp.at[pl.ds(wid * N_FLAT, N_FLAT)])


_cker = functools.partial(
    pl.kernel,
    out_type=jax.ShapeDtypeStruct((NW * N_FLAT,), jnp.float32),
    mesh=_mesh,
    scratch_types=[
        pltpu.VMEM((N_FLAT,), jnp.float32),
        pltpu.VMEM((N_FLAT,), jnp.float32),
        pltpu.VMEM((CH,), jnp.int32),
        pltpu.VMEM((CH,), jnp.int32),
        pltpu.VMEM((CH,), jnp.int32),
        pltpu.VMEM((CH,), jnp.int32),
        pltpu.SemaphoreType.DMA,
        pltpu.SemaphoreType.DMA,
    ],
    compiler_params=pltpu.CompilerParams(needs_layout_passes=False),
)(_cker_body)


def _agg2_body(h1a, h1b, srcf, dstf, zfeat,
               s1a, s1b,
               src_v, dst_v, r0, r1, r2, r3,
               sg0, sg1, sg2, sg3, ss0, ss1, ss2, ss3,
               acc):
    c = lax.axis_index("c")
    s = lax.axis_index("s")
    pltpu.sync_copy(zfeat, acc.at[pl.ds(s * ROWS_PER_TILE, ROWS_PER_TILE)])
    base = s * (C_CH * CH2)
    bufs = (r0, r1, r2, r3)
    sgs = (sg0, sg1, sg2, sg3)
    sss = (ss0, ss1, ss2, ss3)
    plsc.subcore_barrier()

    def gather_feat(eoff, buf, sem):
        @pl.when(c == 0)
        def _():
            pltpu.async_copy(h1a.at[src_v.at[pl.ds(eoff, K)]], buf, sem)

        @pl.when(c == 1)
        def _():
            pltpu.async_copy(h1b.at[src_v.at[pl.ds(eoff, K)]], buf, sem)

    def wait_feat(eoff, buf, sem):
        pltpu.make_async_copy(h1a.at[src_v.at[pl.ds(eoff, K)]], buf, sem).wait()

    def chunk(ci, carry):
        off = base + ci * CH2
        pltpu.sync_copy(srcf.at[pl.ds(off, CH2)], src_v)
        pltpu.sync_copy(dstf.at[pl.ds(off, CH2)], dst_v)
        for r in range(NBUF):
            gather_feat(r * K, bufs[r], sgs[r])

        def grp(g, carry2):
            gbase = g * (NBUF * K)
            for r in range(NBUF):
                eoff = gbase + r * K
                wait_feat(eoff, bufs[r], sgs[r])
                pltpu.async_copy(
                    bufs[r], acc.at[dst_v.at[pl.ds(eoff, K)]], sss[r], add=True)

            @pl.when(g < GRP2 - 1)
            def _():
                for r in range(NBUF):
                    noff = gbase + NBUF * K + r * K
                    pltpu.make_async_copy(
                        bufs[r], acc.at[dst_v.at[pl.ds(0, K)]], sss[r]).wait()
                    gather_feat(noff, bufs[r], sgs[r])

            return carry2

        lax.fori_loop(0, GRP2, grp, carry)
        for r in range(NBUF):
            pltpu.make_async_copy(
                bufs[r], acc.at[dst_v.at[pl.ds(0, K)]], sss[r]).wait()
        return carry

    lax.fori_loop(0, C_CH, chunk, 0)
    plsc.subcore_barrier()

    rsl = pl.ds(s * ROWS_PER_TILE, ROWS_PER_TILE)

    @pl.when(c == 0)
    def _():
        pltpu.sync_copy(acc.at[rsl], s1a.at[rsl])

    @pl.when(c == 1)
    def _():
        pltpu.sync_copy(acc.at[rsl], s1b.at[rsl])


_agg2 = functools.partial(
    pl.kernel,
    out_type=[
        jax.ShapeDtypeStruct((N_ROWS, D_IN), jnp.float32),
        jax.ShapeDtypeStruct((N_ROWS, D_IN), jnp.float32),
    ],
    mesh=_mesh,
    scratch_types=[
        pltpu.VMEM((CH2,), jnp.int32),
        pltpu.VMEM((CH2,), jnp.int32),
        pltpu.VMEM((K, D_IN), jnp.float32),
        pltpu.VMEM((K, D_IN), jnp.float32),
        pltpu.VMEM((K, D_IN), jnp.float32),
        pltpu.VMEM((K, D_IN), jnp.float32),
        pltpu.SemaphoreType.DMA,
        pltpu.SemaphoreType.DMA,
        pltpu.SemaphoreType.DMA,
        pltpu.SemaphoreType.DMA,
        pltpu.SemaphoreType.DMA,
        pltpu.SemaphoreType.DMA,
        pltpu.SemaphoreType.DMA,
        pltpu.SemaphoreType.DMA,
        pltpu.VMEM_SHARED((N_ROWS, D_IN), jnp.float32),
    ],
)(_agg2_body)


RB = 400
GRID = N // RB


def _degw_body(degp_ref, w_ref):
    dsum = jnp.sum(degp_ref[...], axis=0)
    row = lax.broadcasted_iota(jnp.int32, (N_FLAT // 128, 128), 0)
    col = lax.broadcasted_iota(jnp.int32, (N_FLAT // 128, 128), 1)
    nid = row * 128 + col
    w_ref[...] = jnp.where(nid < N, 1.0 / jnp.maximum(dsum, 1.0), 0.0)


def _degw(degp):
    return pl.pallas_call(
        _degw_body,
        out_shape=jax.ShapeDtypeStruct((N_FLAT // 128, 128), jnp.float32),
    )(degp)


def _csum_body(cp_ref, c_ref):
    c_ref[...] = jnp.sum(cp_ref[...], axis=0)


def _csum(cp):
    return pl.pallas_call(
        _csum_body,
        out_shape=jax.ShapeDtypeStruct((N_FLAT // 128, 128), jnp.float32),
    )(cp)


def _dense1_body(x_ref, sa_ref, sb_ref, w_ref, ws_ref, wn_ref, b_ref,
                 z_ref, sum_ref, sq_ref):
    i = pl.program_id(0)
    hn = (sa_ref[...] + sb_ref[...]) * w_ref[...]
    z = (jnp.dot(x_ref[...], ws_ref[...], preferred_element_type=jnp.float32)
         + jnp.dot(hn, wn_ref[...], preferred_element_type=jnp.float32)
         + b_ref[...])
    z_ref[...] = z
    zs = jnp.sum(z, axis=0, keepdims=True)
    z2 = jnp.sum(z * z, axis=0, keepdims=True)

    @pl.when(i == 0)
    def _():
        sum_ref[...] = zs
        sq_ref[...] = z2

    @pl.when(i > 0)
    def _():
        sum_ref[...] += zs
        sq_ref[...] += z2


def _dense1(x, sa, sb, w_col, ws, wn, b):
    d_in = x.shape[1]
    return pl.pallas_call(
        _dense1_body,
        grid=(GRID,),
        in_specs=[
            pl.BlockSpec((RB, d_in), lambda i: (i, 0)),
            pl.BlockSpec((RB, d_in), lambda i: (i, 0)),
            pl.BlockSpec((RB, d_in), lambda i: (i, 0)),
            pl.BlockSpec((RB, 1), lambda i: (i, 0)),
            pl.BlockSpec((d_in, D_H), lambda i: (0, 0)),
            pl.BlockSpec((d_in, D_H), lambda i: (0, 0)),
            pl.BlockSpec((1, D_H), lambda i: (0, 0)),
        ],
        out_specs=[
            pl.BlockSpec((RB, D_H), lambda i: (i, 0)),
            pl.BlockSpec((1, D_H), lambda i: (0, 0)),
            pl.BlockSpec((1, D_H), lambda i: (0, 0)),
        ],
        out_shape=[
            jax.ShapeDtypeStruct((N, D_H), jnp.float32),
            jax.ShapeDtypeStruct((1, D_H), jnp.float32),
            jax.ShapeDtypeStruct((1, D_H), jnp.float32),
        ],
    )(x, sa, sb, w_col, ws, wn, b)


def _dense2_body(ha_ref, hb_ref, sa_ref, sb_ref, w_ref, ws_ref, wn_ref,
                 b_ref, z_ref, sum_ref, sq_ref):
    i = pl.program_id(0)
    r = w_ref[...]
    hna = sa_ref[...] * r
    hnb = sb_ref[...] * r
    ws = ws_ref[...]
    wn = wn_ref[...]
    z = (jnp.dot(ha_ref[...], ws[:D_IN, :], preferred_element_type=jnp.float32)
         + jnp.dot(hb_ref[...], ws[D_IN:, :], preferred_element_type=jnp.float32)
         + jnp.dot(hna, wn[:D_IN, :], preferred_element_type=jnp.float32)
         + jnp.dot(hnb, wn[D_IN:, :], preferred_element_type=jnp.float32)
         + b_ref[...])
    z_ref[...] = z
    zs = jnp.sum(z, axis=0, keepdims=True)
    z2 = jnp.sum(z * z, axis=0, keepdims=True)

    @pl.when(i == 0)
    def _():
        sum_ref[...] = zs
        sq_ref[...] = z2

    @pl.when(i > 0)
    def _():
        sum_ref[...] += zs
        sq_ref[...] += z2


def _dense2(ha, hb, sa, sb, w_col, ws, wn, b):
    return pl.pallas_call(
        _dense2_body,
        grid=(GRID,),
        in_specs=[
            pl.BlockSpec((RB, D_IN), lambda i: (i, 0)),
            pl.BlockSpec((RB, D_IN), lambda i: (i, 0)),
            pl.BlockSpec((RB, D_IN), lambda i: (i, 0)),
            pl.BlockSpec((RB, D_IN), lambda i: (i, 0)),
            pl.BlockSpec((RB, 1), lambda i: (i, 0)),
            pl.BlockSpec((D_H, D_H), lambda i: (0, 0)),
            pl.BlockSpec((D_H, D_H), lambda i: (0, 0)),
            pl.BlockSpec((1, D_H), lambda i: (0, 0)),
        ],
        out_specs=[
            pl.BlockSpec((RB, D_H), lambda i: (i, 0)),
            pl.BlockSpec((1, D_H), lambda i: (0, 0)),
            pl.BlockSpec((1, D_H), lambda i: (0, 0)),
        ],
        out_shape=[
            jax.ShapeDtypeStruct((N, D_H), jnp.float32),
            jax.ShapeDtypeStruct((1, D_H), jnp.float32),
            jax.ShapeDtypeStruct((1, D_H), jnp.float32),
        ],
    )(ha, hb, sa, sb, w_col, ws, wn, b)


def _bnrelu_split_body(z_ref, sum_ref, sq_ref, g_ref, be_ref, ha_ref, hb_ref):
    mu = sum_ref[...] * (1.0 / N)
    var = sq_ref[...] * (1.0 / N) - mu * mu
    inv = lax.rsqrt(var + 1e-5) * g_ref[...]
    h = jnp.maximum((z_ref[...] - mu) * inv + be_ref[...], 0.0)
    ha_ref[...] = h[:, :D_IN]
    hb_ref[...] = h[:, D_IN:]


def _bnrelu_split(z, zsum, zsq, gamma, beta):
    return pl.pallas_call(
        _bnrelu_split_body,
        grid=(GRID,),
        in_specs=[
            pl.BlockSpec((RB, D_H), lambda i: (i, 0)),
            pl.BlockSpec((1, D_H), lambda i: (0, 0)),
            pl.BlockSpec((1, D_H), lambda i: (0, 0)),
            pl.BlockSpec((1, D_H), lambda i: (0, 0)),
            pl.BlockSpec((1, D_H), lambda i: (0, 0)),
        ],
        out_specs=[
            pl.BlockSpec((RB, D_IN), lambda i: (i, 0)),
            pl.BlockSpec((RB, D_IN), lambda i: (i, 0)),
        ],
        out_shape=[
            jax.ShapeDtypeStruct((N_ROWS, D_IN), jnp.float32),
            jax.ShapeDtypeStruct((N_ROWS, D_IN), jnp.float32),
        ],
    )(z, zsum, zsq, gamma, beta)


def _final_body(z_ref, sum_ref, sq_ref, g_ref, be_ref, c_ref,
                ws2_ref, wn2_ref, b2_ref, wlin_ref, blin_ref,
                out_ref, s2_acc, t2_acc):
    i = pl.program_id(0)
    mu = sum_ref[...] * (1.0 / N)
    var = sq_ref[...] * (1.0 / N) - mu * mu
    inv = lax.rsqrt(var + 1e-5) * g_ref[...]
    h = jnp.maximum((z_ref[...] - mu) * inv + be_ref[...], 0.0)
    s2 = jnp.sum(h, axis=0, keepdims=True)
    t2 = jnp.sum(h * c_ref[...], axis=0, keepdims=True)

    @pl.when(i == 0)
    def _():
        s2_acc[...] = s2
        t2_acc[...] = t2

    @pl.when(i > 0)
    def _():
        s2_acc[...] += s2
        t2_acc[...] += t2

    @pl.when(i == GRID - 1)
    def _():
        hg = (jnp.dot(s2_acc[...], ws2_ref[...], preferred_element_type=jnp.float32)
              + jnp.dot(t2_acc[...], wn2_ref[...], preferred_element_type=jnp.float32)
              + float(N) * b2_ref[...])
        out_ref[...] = (jnp.dot(hg, wlin_ref[...], preferred_element_type=jnp.float32)
                        + blin_ref[...])


def _final(z, zsum, zsq, gamma, beta, c_col, ws2, wn2, b2, wlin, blin):
    return pl.pallas_call(
        _final_body,
        grid=(GRID,),
        in_specs=[
            pl.BlockSpec((RB, D_H), lambda i: (i, 0)),
            pl.BlockSpec((1, D_H), lambda i: (0, 0)),
            pl.BlockSpec((1, D_H), lambda i: (0, 0)),
            pl.BlockSpec((1, D_H), lambda i: (0, 0)),
            pl.BlockSpec((1, D_H), lambda i: (0, 0)),
            pl.BlockSpec((RB, 1), lambda i: (i, 0)),
            pl.BlockSpec((D_H, D_H), lambda i: (0, 0)),
            pl.BlockSpec((D_H, D_H), lambda i: (0, 0)),
            pl.BlockSpec((1, D_H), lambda i: (0, 0)),
            pl.BlockSpec((D_H, D_OUT), lambda i: (0, 0)),
            pl.BlockSpec((1, D_OUT), lambda i: (0, 0)),
        ],
        out_specs=pl.BlockSpec((1, D_OUT), lambda i: (0, 0)),
        out_shape=jax.ShapeDtypeStruct((1, D_OUT), jnp.float32),
        scratch_shapes=[
            pltpu.VMEM((1, D_H), jnp.float32),
            pltpu.VMEM((1, D_H), jnp.float32),
        ],
    )(z, zsum, zsq, gamma, beta, c_col, ws2, wn2, b2, wlin, blin)


def kernel(x, edge_index, W_self0, W_neigh0, b0, W_self1, W_neigh1, b1,
           W_self2, W_neigh2, b2, gamma0, beta0, gamma1, beta1, W_lin, b_lin):
    src = edge_index[0]
    dst = edge_index[1]
    pad = E_PAD - E
    srcf = jnp.concatenate([src, jnp.zeros((pad,), jnp.int32)])
    dstf = jnp.concatenate([dst, jnp.full((pad,), N, jnp.int32)])
    zfeat = jnp.zeros((ROWS_PER_TILE, D_IN), jnp.float32)
    zdeg = jnp.zeros((N_FLAT,), jnp.float32)

    s0a, s0b, degp = _agg1(x, srcf, dstf, zfeat, zdeg)
    w2 = _degw(degp.reshape(NW, N_FLAT // 128, 128))
    w_flat = w2.reshape(N_FLAT)
    w_col = w_flat[:N, None]
    cp = _cker(w_flat, srcf, dstf, zdeg)

    z1, z1s, z1q = _dense1(x, s0a, s0b, w_col, W_self0, W_neigh0, b0[None, :])
    h1a, h1b = _bnrelu_split(z1, z1s, z1q, gamma0[None, :], beta0[None, :])

    s1a, s1b = _agg2(h1a, h1b, srcf, dstf, zfeat)
    c2 = _csum(cp.reshape(NW, N_FLAT // 128, 128))
    c_col = c2.reshape(N_FLAT)[:N, None]

    z2, z2s, z2q = _dense2(h1a, h1b, s1a, s1b, w_col,
                           W_self1, W_neigh1, b1[None, :])
    out = _final(z2, z2s, z2q, gamma1[None, :], beta1[None, :], c_col,
                 W_self2, W_neigh2, b2[None, :], W_lin, b_lin[None, :])
    return out
